# baseline (device time: 10835 ns/iter reference)
import jax
import jax.numpy as jnp
from jax import lax
from jax.experimental import pallas as pl
from jax.experimental.pallas import tpu as pltpu

N_DEV = 4
PAD = 160


def kernel(x, dest):
    m_per, n = x.shape
    dest2 = dest.reshape(4, 128)

    def body(x_ref, d_ref, out_ref, pk_ref, rv_ref, dg_ref, xv_ref,
             xs_sem, xr_sem, ds_sem, dr_sem, xcp_sem):
        me = lax.axis_index("i")
        right = lax.rem(me + 1, N_DEV)
        diag = lax.rem(me + 2, N_DEV)
        left = lax.rem(me + 3, N_DEV)

        xcp = pltpu.make_async_copy(x_ref, xv_ref, xcp_sem)
        xcp.start()

        with jax.named_scope("barrier"):
            barrier = pltpu.get_barrier_semaphore()
            for nbr in [left, right, diag]:
                pl.semaphore_signal(
                    barrier, inc=1,
                    device_id=(nbr,), device_id_type=pl.DeviceIdType.MESH,
                )
            pl.semaphore_wait(barrier, 3)

        dg_ref[0, 0:4, :] = d_ref[:]

        d_sends = [(right, 3, 0), (left, 1, 1), (diag, 2, 2)]
        d_rdmas = []
        for dev, q, ss in d_sends:
            r = pltpu.make_async_remote_copy(
                src_ref=dg_ref.at[0], dst_ref=dg_ref.at[q],
                send_sem=ds_sem.at[ss], recv_sem=dr_sem.at[q],
                device_id=(dev,), device_id_type=pl.DeviceIdType.MESH,
            )
            r.start()
            d_rdmas.append(r)

        xcp.wait()
        xbf = xv_ref[:].astype(jnp.bfloat16)
        a_io = lax.broadcasted_iota(jnp.int32, (128, 128), 0)
        b_io = lax.broadcasted_iota(jnp.int32, (128, 128), 1)
        tri = (a_io <= b_io).astype(jnp.float32)
        r_io = lax.broadcasted_iota(jnp.int32, (4, 4), 0)
        s_io = lax.broadcasted_iota(jnp.int32, (4, 4), 1)
        tex4 = (s_io < r_io).astype(jnp.float32)
        kio = lax.broadcasted_iota(jnp.int32, (PAD, 128), 0)

        def pack_for(r_target):
            mloc = (d_ref[:] == r_target).astype(jnp.float32)
            c1 = lax.dot_general(
                mloc, tri, (((1,), (0,)), ((), ())),
                preferred_element_type=jnp.float32,
            )
            offs = lax.dot_general(
                tex4, c1[:, 127:128], (((1,), (0,)), ((), ())),
                preferred_element_type=jnp.float32,
            )
            cm = ((c1 + offs) * mloc).astype(jnp.int32)
            acc = jnp.zeros((PAD, n), jnp.float32)
            for a in range(4):
                pb = (cm[a:a + 1, :] == kio + 1).astype(jnp.bfloat16)
                acc = acc + lax.dot_general(
                    pb, xbf[128 * a:128 * (a + 1), :],
                    (((1,), (0,)), ((), ())),
                    preferred_element_type=jnp.float32,
                )
            return acc.astype(jnp.bfloat16)

        x_sends = [(diag, 2, 2), (right, 3, 1), (left, 1, 3)]
        x_rdmas = []
        for dev, q, rel in x_sends:
            with jax.named_scope(f"pack#rel={rel}"):
                pk_ref[rel] = pack_for(lax.rem(me + rel, N_DEV))
                r = pltpu.make_async_remote_copy(
                    src_ref=pk_ref.at[rel], dst_ref=rv_ref.at[q],
                    send_sem=xs_sem.at[rel], recv_sem=xr_sem.at[q],
                    device_id=(dev,), device_id_type=pl.DeviceIdType.MESH,
                )
                r.start()
                x_rdmas.append(r)
        with jax.named_scope("pack#rel=0"):
            rv_ref[0] = pack_for(me)

        with jax.named_scope("dwait"):
            for r in d_rdmas:
                r.wait_recv()

        with jax.named_scope("counts"):
            counts = [
                jnp.sum(
                    (dg_ref[q, 0:4, :] == me).astype(jnp.float32)
                ).astype(jnp.int32)
                for q in range(N_DEV)
            ]
            origin = [lax.rem(me + q, N_DEV) for q in range(N_DEV)]
            base = [
                sum(
                    jnp.where(origin[p] < origin[q], counts[p], 0)
                    for p in range(N_DEV)
                    if p != q
                )
                for q in range(N_DEV)
            ]

        kp = lax.broadcasted_iota(jnp.int32, (m_per, PAD), 0)
        kk = lax.broadcasted_iota(jnp.int32, (m_per, PAD), 1)
        kd = kp - kk

        def place(q, acc):
            sel = (kd == base[q]).astype(jnp.bfloat16)
            return acc + lax.dot_general(
                sel, rv_ref[q], (((1,), (0,)), ((), ())),
                preferred_element_type=jnp.float32,
            )

        acc = jnp.zeros((m_per, n), jnp.float32)
        with jax.named_scope("place#q=0"):
            acc = place(0, acc)
        with jax.named_scope("wait#q=3"):
            x_rdmas[1].wait_recv()
        with jax.named_scope("place#q=3"):
            acc = place(3, acc)
        with jax.named_scope("wait#q=1"):
            x_rdmas[2].wait_recv()
        with jax.named_scope("place#q=1"):
            acc = place(1, acc)
        with jax.named_scope("wait#q=2"):
            x_rdmas[0].wait_recv()
        with jax.named_scope("place#q=2"):
            acc = place(2, acc)
        with jax.named_scope("store"):
            out_ref[:] = acc.astype(jnp.bfloat16)

        with jax.named_scope("drain"):
            for r in d_rdmas:
                r.wait_send()
            for r in x_rdmas:
                r.wait_send()

    return pl.pallas_call(
        body,
        out_shape=jax.ShapeDtypeStruct((m_per, n), jnp.bfloat16),
        in_specs=[
            pl.BlockSpec(memory_space=pl.ANY),
            pl.BlockSpec(memory_space=pltpu.VMEM),
        ],
        out_specs=pl.BlockSpec(memory_space=pltpu.VMEM),
        scratch_shapes=[
            pltpu.VMEM((N_DEV, PAD, n), jnp.bfloat16),
            pltpu.VMEM((N_DEV, PAD, n), jnp.bfloat16),
            pltpu.VMEM((N_DEV, 8, 128), jnp.int32),
            pltpu.VMEM((m_per, n), jnp.float32),
            pltpu.SemaphoreType.DMA((N_DEV,)),
            pltpu.SemaphoreType.DMA((N_DEV,)),
            pltpu.SemaphoreType.DMA((3,)),
            pltpu.SemaphoreType.DMA((N_DEV,)),
            pltpu.SemaphoreType.DMA,
        ],
        compiler_params=pltpu.CompilerParams(collective_id=0),
    )(x, dest2)


# device time: 10762 ns/iter; 1.0068x vs baseline; 1.0068x over previous
import jax
import jax.numpy as jnp
from jax import lax
from jax.experimental import pallas as pl
from jax.experimental.pallas import tpu as pltpu

N_DEV = 4
PAD = 160


def kernel(x, dest):
    m_per, n = x.shape
    dest2 = dest.reshape(4, 128)

    def body(x_ref, d_ref, out_ref, pk_ref, rv_ref, dg_ref, m_ref, xv_ref,
             xs_sem, xr_sem, ds_sem, dr_sem, xcp_sem):
        me = lax.axis_index("i")
        right = lax.rem(me + 1, N_DEV)
        diag = lax.rem(me + 2, N_DEV)
        left = lax.rem(me + 3, N_DEV)

        xcp = pltpu.make_async_copy(x_ref, xv_ref, xcp_sem)
        xcp.start()

        barrier = pltpu.get_barrier_semaphore()
        for nbr in [left, right, diag]:
            pl.semaphore_signal(
                barrier, inc=1,
                device_id=(nbr,), device_id_type=pl.DeviceIdType.MESH,
            )
        pl.semaphore_wait(barrier, 3)

        dg_ref[0, 0:4, :] = d_ref[:]

        d_sends = [(right, 3, 0), (left, 1, 1), (diag, 2, 2)]
        d_rdmas = []
        for dev, q, ss in d_sends:
            r = pltpu.make_async_remote_copy(
                src_ref=dg_ref.at[0], dst_ref=dg_ref.at[q],
                send_sem=ds_sem.at[ss], recv_sem=dr_sem.at[q],
                device_id=(dev,), device_id_type=pl.DeviceIdType.MESH,
            )
            r.start()
            d_rdmas.append(r)

        xcp.wait()
        xbf = xv_ref[:].astype(jnp.bfloat16)
        for g in range(N_DEV):
            m_ref[4 * g:4 * g + 4, :] = (
                d_ref[:] == lax.rem(me + g, N_DEV)
            ).astype(jnp.float32)
        mall = m_ref[:]
        a_io = lax.broadcasted_iota(jnp.int32, (128, 128), 0)
        b_io = lax.broadcasted_iota(jnp.int32, (128, 128), 1)
        tri = (a_io <= b_io).astype(jnp.float32)
        c1 = lax.dot_general(
            mall, tri, (((1,), (0,)), ((), ())),
            preferred_element_type=jnp.float32,
        )
        i_io = lax.broadcasted_iota(jnp.int32, (16, 16), 0)
        p_io = lax.broadcasted_iota(jnp.int32, (16, 16), 1)
        texg = ((p_io // 4 == i_io // 4) & (p_io < i_io)).astype(jnp.float32)
        offs = lax.dot_general(
            texg, c1[:, 127:128], (((1,), (0,)), ((), ())),
            preferred_element_type=jnp.float32,
        )
        cm = ((c1 + offs) * mall).astype(jnp.int32)

        kio = lax.broadcasted_iota(jnp.int32, (PAD, 128), 0)

        def pack_for(g, dst_slot_ref):
            acc = None
            for a in range(4):
                pb = (cm[4 * g + a:4 * g + a + 1, :] == kio + 1).astype(
                    jnp.bfloat16
                )
                d = lax.dot_general(
                    pb, xbf[128 * a:128 * (a + 1), :],
                    (((1,), (0,)), ((), ())),
                    preferred_element_type=jnp.float32,
                )
                acc = d if acc is None else acc + d
            dst_slot_ref[...] = acc.astype(jnp.bfloat16)

        x_sends = [(diag, 2, 2), (right, 3, 1), (left, 1, 3)]
        x_rdmas = []
        for dev, q, g in x_sends:
            pack_for(g, pk_ref.at[g])
            r = pltpu.make_async_remote_copy(
                src_ref=pk_ref.at[g], dst_ref=rv_ref.at[q],
                send_sem=xs_sem.at[g], recv_sem=xr_sem.at[q],
                device_id=(dev,), device_id_type=pl.DeviceIdType.MESH,
            )
            r.start()
            x_rdmas.append(r)
        pack_for(0, rv_ref.at[0])

        for r in d_rdmas:
            r.wait_recv()

        counts = [
            jnp.sum((dg_ref[q, 0:4, :] == me).astype(jnp.float32)).astype(
                jnp.int32
            )
            for q in range(N_DEV)
        ]
        origin = [lax.rem(me + q, N_DEV) for q in range(N_DEV)]
        base = [
            sum(
                jnp.where(origin[p] < origin[q], counts[p], 0)
                for p in range(N_DEV)
                if p != q
            )
            for q in range(N_DEV)
        ]

        kp = lax.broadcasted_iota(jnp.int32, (m_per, PAD), 0)
        kk = lax.broadcasted_iota(jnp.int32, (m_per, PAD), 1)
        kd = kp - kk
        sel = {
            q: (kd == base[q]).astype(jnp.bfloat16)
            for q in range(N_DEV)
        }

        def place(q, acc):
            d = lax.dot_general(
                sel[q], rv_ref[q], (((1,), (0,)), ((), ())),
                preferred_element_type=jnp.float32,
            )
            return d if acc is None else acc + d

        acc = place(0, None)
        x_rdmas[1].wait_recv()
        acc = place(3, acc)
        x_rdmas[2].wait_recv()
        acc = place(1, acc)
        x_rdmas[0].wait_recv()
        acc = place(2, acc)
        out_ref[:] = acc.astype(jnp.bfloat16)

        for r in d_rdmas:
            r.wait_send()
        for r in x_rdmas:
            r.wait_send()

    return pl.pallas_call(
        body,
        out_shape=jax.ShapeDtypeStruct((m_per, n), jnp.bfloat16),
        in_specs=[
            pl.BlockSpec(memory_space=pl.ANY),
            pl.BlockSpec(memory_space=pltpu.VMEM),
        ],
        out_specs=pl.BlockSpec(memory_space=pltpu.VMEM),
        scratch_shapes=[
            pltpu.VMEM((N_DEV, PAD, n), jnp.bfloat16),
            pltpu.VMEM((N_DEV, PAD, n), jnp.bfloat16),
            pltpu.VMEM((N_DEV, 8, 128), jnp.int32),
            pltpu.VMEM((16, 128), jnp.float32),
            pltpu.VMEM((m_per, n), jnp.float32),
            pltpu.SemaphoreType.DMA((N_DEV,)),
            pltpu.SemaphoreType.DMA((N_DEV,)),
            pltpu.SemaphoreType.DMA((3,)),
            pltpu.SemaphoreType.DMA((N_DEV,)),
            pltpu.SemaphoreType.DMA,
        ],
        compiler_params=pltpu.CompilerParams(collective_id=0),
    )(x, dest2)
